# Initial kernel scaffold; baseline (speedup 1.0000x reference)
import jax
import jax.numpy as jnp
from jax.experimental import pallas as pl


def kernel(input, target, W):
    raise NotImplementedError("write your pallas kernel here")


# speedup vs baseline: 34.4091x; 34.4091x over previous
import functools

import jax
import jax.numpy as jnp
from jax import lax
from jax.experimental import pallas as pl
from jax.experimental.pallas import tpu as pltpu
from jax.experimental.pallas import tpu_sc as plsc

_FEAT = 64
_CDIM = 100000
_B = 1024
_LAMB = max(5.0, 1500.0 / (1.0 + 0.1 * 1))
_COEF = 1.0 / (1.0 + _LAMB)
_RT2H = 0.7071067811865476
_NC = 2
_NS = 16
_CBLK = 128
_NBLK = _B // _CBLK


def _sc_gather_body(w_hbm, t_hbm, out_hbm, t_v, idx_v, row_v, sem):
    wid = lax.axis_index("s") * _NC + lax.axis_index("c")
    pltpu.sync_copy(t_hbm, t_v)
    for ff in range(2):
        f = wid * 2 + ff
        base = f * _CDIM
        for j in range(_NBLK):
            for kk in range(8):
                sl = pl.ds(kk * 16, 16)
                idx_v[j, sl] = t_v[j, sl] + base
        cps = [
            pltpu.async_copy(
                w_hbm.at[idx_v.at[j]], row_v.at[pl.ds(j * _CBLK, _CBLK)], sem
            )
            for j in range(_NBLK)
        ]
        for cp in cps:
            cp.wait()
        pltpu.sync_copy(row_v, out_hbm.at[f])


@functools.cache
def _sc_gather():
    return pl.kernel(
        _sc_gather_body,
        mesh=plsc.VectorSubcoreMesh(core_axis_name="c", subcore_axis_name="s"),
        out_type=jax.ShapeDtypeStruct((_FEAT, _B), jnp.float32),
        scratch_types=[
            pltpu.VMEM((_NBLK, _CBLK), jnp.int32),
            pltpu.VMEM((_NBLK, _CBLK), jnp.int32),
            pltpu.VMEM((_B,), jnp.float32),
            pltpu.SemaphoreType.DMA,
        ],
    )


def _tc_body(x_ref, wt_ref, tcol_ref, trow_ref, out_ref, acc_ref):
    j = pl.program_id(0)
    f32 = jnp.float32
    x = x_ref[...]
    wt = wt_ref[...]
    xlen = jnp.sqrt(jnp.sum(x * x, axis=1, keepdims=True))
    wn = jnp.sqrt(jnp.sum(wt * wt, axis=0, keepdims=True))
    s = jnp.dot(x, wt, preferred_element_type=f32)
    c = jnp.clip(s / (xlen * wn), -1.0, 1.0)
    c2 = c * c
    cos_m = 8.0 * c2 * c2 - 8.0 * c2 + 1.0
    k = ((c <= _RT2H).astype(f32) + (c <= 0.0).astype(f32)
         + (c <= -_RT2H).astype(f32) + (c <= -1.0).astype(f32))
    k_even = (c > _RT2H) | ((c <= 0.0) & (c > -_RT2H)) | (c <= -1.0)
    phi = jnp.where(k_even, cos_m, -cos_m) - 2.0 * k
    cos_sc = c * xlen
    phi_sc = phi * xlen
    m = (tcol_ref[...] == trow_ref[...]).astype(f32)
    outm = cos_sc - m * cos_sc * _COEF + m * phi_sc * _COEF
    cmax = jnp.max(outm, axis=0, keepdims=True)
    lse = cmax + jnp.log(jnp.sum(jnp.exp(outm - cmax), axis=0, keepdims=True))
    row_i = lax.broadcasted_iota(jnp.int32, (_B, _CBLK), 0)
    col_i = lax.broadcasted_iota(jnp.int32, (_B, _CBLK), 1) + j * _CBLK
    tr = jnp.sum(jnp.where(row_i == col_i, outm, 0.0))

    @pl.when(j == 0)
    def _init():
        acc_ref[0] = 0.0
        acc_ref[1] = 0.0

    acc_ref[0] += tr
    acc_ref[1] += jnp.sum(lse)

    @pl.when(j == pl.num_programs(0) - 1)
    def _fin():
        out_ref[0, 0] = -(acc_ref[0] - acc_ref[1]) / _B


_tc_loss = pl.pallas_call(
    _tc_body,
    grid=(_NBLK,),
    in_specs=[
        pl.BlockSpec((_B, _FEAT), lambda j: (0, 0)),
        pl.BlockSpec((_FEAT, _CBLK), lambda j: (0, j)),
        pl.BlockSpec((_B, 1), lambda j: (0, 0)),
        pl.BlockSpec((1, _CBLK), lambda j: (0, j)),
    ],
    out_specs=pl.BlockSpec(memory_space=pltpu.SMEM),
    out_shape=jax.ShapeDtypeStruct((1, 1), jnp.float32),
    scratch_shapes=[pltpu.SMEM((2,), jnp.float32)],
)


def kernel(input, target, W):
    wt = _sc_gather()(W.reshape(-1), target.reshape(_NBLK, _CBLK))
    loss = _tc_loss(input, wt, target.reshape(_B, 1), target.reshape(1, _B))
    return loss.reshape(())
